# baseline (speedup 1.0000x reference)
import functools

import jax
import jax.numpy as jnp
from jax import lax
from jax.experimental import pallas as pl
from jax.experimental.pallas import tpu as pltpu
from jax.experimental.pallas import tpu_sc as plsc

D_MODEL = 768
D_HID = 1536
NE = 8
TB = 256
BLK = 256
T = 2048
CAP = T
SPB = CAP // BLK
NB = NE * SPB
R = NE * CAP
NEG = -1e30

NW = 32
L = 16
TPW = T // NW
NCK = 4
CK = TPW // NCK


def _router_body(x_ref, gw_ref, p1_ref, p2_ref, wa_ref, wb_ref,
                 val_ref, xmap_ref, offs_ref):
    logits = jnp.dot(x_ref[...], gw_ref[...], preferred_element_type=jnp.float32)
    lane = lax.broadcasted_iota(jnp.int32, (TB, NE), 1)
    m1 = jnp.max(logits, axis=1, keepdims=True)
    i1 = jnp.min(jnp.where(logits == m1, lane, NE), axis=1, keepdims=True)
    l2 = jnp.where(lane == i1, NEG, logits)
    m2 = jnp.max(l2, axis=1, keepdims=True)
    i2 = jnp.min(jnp.where(l2 == m2, lane, NE), axis=1, keepdims=True)
    e2 = jnp.exp(m2 - m1)
    s = 1.0 + e2

    @pl.when(pl.program_id(0) == 0)
    def _():
        offs_ref[...] = jnp.zeros_like(offs_ref)

    pairmask = jnp.where((lane == i1) | (lane == i2), 1.0, 0.0)
    row = lax.broadcasted_iota(jnp.int32, (TB, TB), 0)
    col = lax.broadcasted_iota(jnp.int32, (TB, TB), 1)
    ltri = jnp.where(col < row, 1.0, 0.0)
    rank = jnp.dot(ltri, pairmask, preferred_element_type=jnp.float32)
    rank = rank + offs_ref[...]
    r1 = jnp.sum(jnp.where(lane == i1, rank, 0.0), axis=1,
                 keepdims=True).astype(jnp.int32)
    r2 = jnp.sum(jnp.where(lane == i2, rank, 0.0), axis=1,
                 keepdims=True).astype(jnp.int32)
    p1_ref[...] = i1 * CAP + r1
    p2_ref[...] = i2 * CAP + r2
    wa_ref[...] = jnp.broadcast_to(1.0 / s, (TB, L))
    wb_ref[...] = jnp.broadcast_to(e2 / s, (TB, L))
    offs = offs_ref[...] + jnp.sum(pairmask, axis=0, keepdims=True)
    offs_ref[...] = offs

    @pl.when(pl.program_id(0) == pl.num_programs(0) - 1)
    def _():
        nblk = jnp.floor((offs + (BLK - 1)) * (1.0 / BLK))
        siota = lax.broadcasted_iota(jnp.int32, (SPB, NE), 0).astype(jnp.float32)
        valid_c = jnp.where(siota * BLK < jnp.broadcast_to(offs, (SPB, NE)),
                            1.0, 0.0)
        smax = jnp.maximum(nblk - 1.0, 0.0)
        xmap_c = jnp.minimum(siota, jnp.broadcast_to(smax, (SPB, NE)))
        r8 = lax.broadcasted_iota(jnp.int32, (NE, NE), 0)
        c8 = lax.broadcasted_iota(jnp.int32, (NE, NE), 1)
        eye = jnp.where(r8 == c8, 1.0, 0.0)
        tdims = (((0,), (0,)), ((), ()))
        valid_r = lax.dot_general(valid_c, eye, tdims,
                                  preferred_element_type=jnp.float32)
        xmap_r = lax.dot_general(xmap_c, eye, tdims,
                                 preferred_element_type=jnp.float32)
        val_ref[...] = valid_r.astype(jnp.int32)
        xmap_ref[...] = (xmap_r + (r8 * SPB).astype(jnp.float32)).astype(jnp.int32)


def _make_dispatch():
    @functools.partial(
        pl.kernel,
        mesh=plsc.VectorSubcoreMesh(core_axis_name="c", subcore_axis_name="s"),
        out_type=jax.ShapeDtypeStruct((R, D_MODEL), jnp.float32),
        scratch_types=[
            pltpu.VMEM((TPW,), jnp.int32),
            pltpu.VMEM((TPW,), jnp.int32),
            pltpu.VMEM((TPW, D_MODEL), jnp.float32),
            pltpu.SemaphoreType.DMA,
            pltpu.SemaphoreType.DMA,
        ],
    )
    def dispatch(x_hbm, pos1_hbm, pos2_hbm, sx_hbm, idx1_v, idx2_v, rows_v,
                 sem1, sem2):
        wid = lax.axis_index("s") * 2 + lax.axis_index("c")
        base = wid * TPW
        pltpu.sync_copy(pos1_hbm.at[pl.ds(base, TPW)], idx1_v)
        pltpu.sync_copy(pos2_hbm.at[pl.ds(base, TPW)], idx2_v)
        pltpu.sync_copy(x_hbm.at[pl.ds(base, TPW)], rows_v)
        cp1 = pltpu.async_copy(rows_v, sx_hbm.at[idx1_v], sem1)
        cp2 = pltpu.async_copy(rows_v, sx_hbm.at[idx2_v], sem2)
        cp1.wait()
        cp2.wait()

    return dispatch


def _gemm_body(xmap_ref, val_ref, xs_ref, w1_ref, b1_ref, w2_ref, b2_ref,
               o_ref, acc_ref):
    b = pl.program_id(0)
    c = pl.program_id(1)

    @pl.when(val_ref[b] != 0)
    def _():
        h = jnp.dot(xs_ref[...], w1_ref[0], preferred_element_type=jnp.float32)
        h = h + b1_ref[0]
        h = 0.5 * h * (1.0 + lax.erf(h * 0.7071067811865476))
        part = jnp.dot(h, w2_ref[0], preferred_element_type=jnp.float32)

        @pl.when(c == 0)
        def _():
            acc_ref[...] = part

        @pl.when(c == 1)
        def _():
            o_ref[...] = acc_ref[...] + part + b2_ref[0]


def _make_combine():
    @functools.partial(
        pl.kernel,
        mesh=plsc.VectorSubcoreMesh(core_axis_name="c", subcore_axis_name="s"),
        out_type=jax.ShapeDtypeStruct((T, D_MODEL), jnp.float32),
        scratch_types=[
            pltpu.VMEM((TPW,), jnp.int32),
            pltpu.VMEM((TPW,), jnp.int32),
            pltpu.VMEM((TPW, L), jnp.float32),
            pltpu.VMEM((TPW, L), jnp.float32),
            pltpu.VMEM((TPW, D_MODEL), jnp.float32),
            pltpu.VMEM((TPW, D_MODEL), jnp.float32),
            pltpu.SemaphoreType.DMA,
            pltpu.SemaphoreType.DMA,
        ],
    )
    def combine(y_hbm, pos1_hbm, pos2_hbm, wa_hbm, wb_hbm, out_hbm,
                idx1_v, idx2_v, wa_v, wb_v, buf1, buf2, sem1, sem2):
        wid = lax.axis_index("s") * 2 + lax.axis_index("c")
        base = wid * TPW
        pltpu.sync_copy(pos1_hbm.at[pl.ds(base, TPW)], idx1_v)
        pltpu.sync_copy(pos2_hbm.at[pl.ds(base, TPW)], idx2_v)
        pltpu.sync_copy(wa_hbm.at[pl.ds(base, TPW)], wa_v)
        pltpu.sync_copy(wb_hbm.at[pl.ds(base, TPW)], wb_v)
        cp1 = pltpu.async_copy(y_hbm.at[idx1_v], buf1, sem1)
        cp2 = pltpu.async_copy(y_hbm.at[idx2_v], buf2, sem2)
        cp1.wait()
        cp2.wait()

        def body(t, carry):
            wa = wa_v[t, :]
            wb = wb_v[t, :]
            for j in range(D_MODEL // L):
                sl = pl.ds(j * L, L)
                buf1[t, sl] = wa * buf1[t, sl] + wb * buf2[t, sl]
            return carry

        lax.fori_loop(0, TPW, body, 0)
        pltpu.sync_copy(buf1, out_hbm.at[pl.ds(base, TPW)])

    return combine


def kernel(x, gate_w, w1, b1, w2, b2):
    B = x.shape[0]
    xf = x.reshape(T, D_MODEL)
    ntb = T // TB
    shp = jax.ShapeDtypeStruct

    p1o, p2o, wao, wbo, valo, xmapo = pl.pallas_call(
        _router_body,
        grid=(ntb,),
        in_specs=[
            pl.BlockSpec((TB, D_MODEL), lambda t: (t, 0)),
            pl.BlockSpec((D_MODEL, NE), lambda t: (0, 0)),
        ],
        out_specs=[
            pl.BlockSpec((TB, 1), lambda t: (t, 0)),
            pl.BlockSpec((TB, 1), lambda t: (t, 0)),
            pl.BlockSpec((TB, L), lambda t: (t, 0)),
            pl.BlockSpec((TB, L), lambda t: (t, 0)),
            pl.BlockSpec((NE, SPB), lambda t: (0, 0)),
            pl.BlockSpec((NE, SPB), lambda t: (0, 0)),
        ],
        out_shape=[
            shp((T, 1), jnp.int32), shp((T, 1), jnp.int32),
            shp((T, L), jnp.float32), shp((T, L), jnp.float32),
            shp((NE, SPB), jnp.int32), shp((NE, SPB), jnp.int32),
        ],
        scratch_shapes=[pltpu.VMEM((1, NE), jnp.float32)],
    )(xf, gate_w)

    pos1 = p1o.reshape(T)
    pos2 = p2o.reshape(T)
    sorted_x = _make_dispatch()(xf, pos1, pos2)

    hc = D_HID // 2
    grid_spec = pltpu.PrefetchScalarGridSpec(
        num_scalar_prefetch=2,
        grid=(NB, 2),
        in_specs=[
            pl.BlockSpec((BLK, D_MODEL), lambda b, c, xm, val: (xm[b], 0)),
            pl.BlockSpec((1, D_MODEL, hc), lambda b, c, xm, val: (b // SPB, 0, c)),
            pl.BlockSpec((1, 1, hc), lambda b, c, xm, val: (b // SPB, 0, c)),
            pl.BlockSpec((1, hc, D_MODEL), lambda b, c, xm, val: (b // SPB, c, 0)),
            pl.BlockSpec((1, 1, D_MODEL), lambda b, c, xm, val: (b // SPB, 0, 0)),
        ],
        out_specs=pl.BlockSpec((BLK, D_MODEL), lambda b, c, xm, val: (xm[b], 0)),
        scratch_shapes=[pltpu.VMEM((BLK, D_MODEL), jnp.float32)],
    )
    y_sorted = pl.pallas_call(
        _gemm_body,
        grid_spec=grid_spec,
        out_shape=shp((R, D_MODEL), jnp.float32),
    )(xmapo.reshape(NB), valo.reshape(NB), sorted_x, w1, b1, w2, b2)

    out = _make_combine()(y_sorted, pos1, pos2, wao, wbo)
    return out.reshape(B, T, D_MODEL)


# speedup vs baseline: 2.0493x; 2.0493x over previous
import functools

import jax
import jax.numpy as jnp
from jax import lax
from jax.experimental import pallas as pl
from jax.experimental.pallas import tpu as pltpu
from jax.experimental.pallas import tpu_sc as plsc

D_MODEL = 768
D_HID = 1536
NE = 8
TB = 256
BLK = 256
T = 2048
CAP = T
SPB = CAP // BLK
NB = NE * SPB
R = NE * CAP
NEG = -1e30

NW = 32
L = 16
TPW = T // NW
NCK = 4
CK = TPW // NCK


def _router_body(x_ref, gw_ref, p1_ref, p2_ref, wa_ref, wb_ref,
                 val_ref, xmap_ref, offs_ref):
    logits = jnp.dot(x_ref[...], gw_ref[...], preferred_element_type=jnp.float32)
    lane = lax.broadcasted_iota(jnp.int32, (TB, NE), 1)
    m1 = jnp.max(logits, axis=1, keepdims=True)
    i1 = jnp.min(jnp.where(logits == m1, lane, NE), axis=1, keepdims=True)
    l2 = jnp.where(lane == i1, NEG, logits)
    m2 = jnp.max(l2, axis=1, keepdims=True)
    i2 = jnp.min(jnp.where(l2 == m2, lane, NE), axis=1, keepdims=True)
    e2 = jnp.exp(m2 - m1)
    s = 1.0 + e2

    @pl.when(pl.program_id(0) == 0)
    def _():
        offs_ref[...] = jnp.zeros_like(offs_ref)

    pairmask = jnp.where((lane == i1) | (lane == i2), 1.0, 0.0)
    row = lax.broadcasted_iota(jnp.int32, (TB, TB), 0)
    col = lax.broadcasted_iota(jnp.int32, (TB, TB), 1)
    ltri = jnp.where(col < row, 1.0, 0.0)
    rank = jnp.dot(ltri, pairmask, preferred_element_type=jnp.float32)
    rank = rank + offs_ref[...]
    r1 = jnp.sum(jnp.where(lane == i1, rank, 0.0), axis=1,
                 keepdims=True).astype(jnp.int32)
    r2 = jnp.sum(jnp.where(lane == i2, rank, 0.0), axis=1,
                 keepdims=True).astype(jnp.int32)
    p1_ref[...] = i1 * CAP + r1
    p2_ref[...] = i2 * CAP + r2
    wa_ref[...] = jnp.broadcast_to(1.0 / s, (TB, L))
    wb_ref[...] = jnp.broadcast_to(e2 / s, (TB, L))
    offs = offs_ref[...] + jnp.sum(pairmask, axis=0, keepdims=True)
    offs_ref[...] = offs

    @pl.when(pl.program_id(0) == pl.num_programs(0) - 1)
    def _():
        nblk = jnp.floor((offs + (BLK - 1)) * (1.0 / BLK))
        siota = lax.broadcasted_iota(jnp.int32, (SPB, NE), 0).astype(jnp.float32)
        valid_c = jnp.where(siota * BLK < jnp.broadcast_to(offs, (SPB, NE)),
                            1.0, 0.0)
        smax = jnp.maximum(nblk - 1.0, 0.0)
        xmap_c = jnp.minimum(siota, jnp.broadcast_to(smax, (SPB, NE)))
        r8 = lax.broadcasted_iota(jnp.int32, (NE, NE), 0)
        c8 = lax.broadcasted_iota(jnp.int32, (NE, NE), 1)
        eye = jnp.where(r8 == c8, 1.0, 0.0)
        tdims = (((0,), (0,)), ((), ()))
        valid_r = lax.dot_general(valid_c, eye, tdims,
                                  preferred_element_type=jnp.float32)
        xmap_r = lax.dot_general(xmap_c, eye, tdims,
                                 preferred_element_type=jnp.float32)
        val_ref[...] = valid_r.astype(jnp.int32)
        xmap_ref[...] = (xmap_r + (r8 * SPB).astype(jnp.float32)).astype(jnp.int32)


def _make_dispatch():
    @functools.partial(
        pl.kernel,
        mesh=plsc.VectorSubcoreMesh(core_axis_name="c", subcore_axis_name="s"),
        out_type=jax.ShapeDtypeStruct((R, D_MODEL), jnp.float32),
        scratch_types=[
            pltpu.VMEM((TPW,), jnp.int32),
            pltpu.VMEM((TPW,), jnp.int32),
            pltpu.VMEM((TPW, D_MODEL), jnp.float32),
            pltpu.SemaphoreType.DMA,
            pltpu.SemaphoreType.DMA,
        ],
    )
    def dispatch(x_hbm, pos1_hbm, pos2_hbm, sx_hbm, idx1_v, idx2_v, rows_v,
                 sem1, sem2):
        wid = lax.axis_index("s") * 2 + lax.axis_index("c")
        base = wid * TPW
        pltpu.sync_copy(pos1_hbm.at[pl.ds(base, TPW)], idx1_v)
        pltpu.sync_copy(pos2_hbm.at[pl.ds(base, TPW)], idx2_v)
        pltpu.sync_copy(x_hbm.at[pl.ds(base, TPW)], rows_v)
        cp1 = pltpu.async_copy(rows_v, sx_hbm.at[idx1_v], sem1)
        cp2 = pltpu.async_copy(rows_v, sx_hbm.at[idx2_v], sem2)
        cp1.wait()
        cp2.wait()

    return dispatch


def _gemm_body(xmap_ref, val_ref, xs_ref, w1_ref, b1_ref, w2_ref, b2_ref, o_ref):
    b = pl.program_id(0)

    @pl.when(val_ref[b] != 0)
    def _():
        h = jnp.dot(xs_ref[...], w1_ref[0], preferred_element_type=jnp.float32)
        h = h + b1_ref[0]
        h = 0.5 * h * (1.0 + lax.erf(h * 0.7071067811865476))
        o_ref[...] = jnp.dot(h, w2_ref[0],
                             preferred_element_type=jnp.float32) + b2_ref[0]


def _make_combine():
    @functools.partial(
        pl.kernel,
        mesh=plsc.VectorSubcoreMesh(core_axis_name="c", subcore_axis_name="s"),
        out_type=jax.ShapeDtypeStruct((T, D_MODEL), jnp.float32),
        scratch_types=[
            pltpu.VMEM((TPW,), jnp.int32),
            pltpu.VMEM((TPW,), jnp.int32),
            pltpu.VMEM((TPW, L), jnp.float32),
            pltpu.VMEM((TPW, L), jnp.float32),
            pltpu.VMEM((TPW, D_MODEL), jnp.float32),
            pltpu.VMEM((TPW, D_MODEL), jnp.float32),
            pltpu.SemaphoreType.DMA,
            pltpu.SemaphoreType.DMA,
        ],
    )
    def combine(y_hbm, pos1_hbm, pos2_hbm, wa_hbm, wb_hbm, out_hbm,
                idx1_v, idx2_v, wa_v, wb_v, buf1, buf2, sem1, sem2):
        wid = lax.axis_index("s") * 2 + lax.axis_index("c")
        base = wid * TPW
        pltpu.sync_copy(pos1_hbm.at[pl.ds(base, TPW)], idx1_v)
        pltpu.sync_copy(pos2_hbm.at[pl.ds(base, TPW)], idx2_v)
        pltpu.sync_copy(wa_hbm.at[pl.ds(base, TPW)], wa_v)
        pltpu.sync_copy(wb_hbm.at[pl.ds(base, TPW)], wb_v)
        cp1 = pltpu.async_copy(y_hbm.at[idx1_v], buf1, sem1)
        cp2 = pltpu.async_copy(y_hbm.at[idx2_v], buf2, sem2)
        cp1.wait()
        cp2.wait()

        def body(t, carry):
            wa = wa_v[t, :]
            wb = wb_v[t, :]
            for j in range(D_MODEL // L):
                sl = pl.ds(j * L, L)
                buf1[t, sl] = wa * buf1[t, sl] + wb * buf2[t, sl]
            return carry

        lax.fori_loop(0, TPW, body, 0)
        pltpu.sync_copy(buf1, out_hbm.at[pl.ds(base, TPW)])

    return combine


def kernel(x, gate_w, w1, b1, w2, b2):
    B = x.shape[0]
    xf = x.reshape(T, D_MODEL)
    ntb = T // TB
    shp = jax.ShapeDtypeStruct

    p1o, p2o, wao, wbo, valo, xmapo = pl.pallas_call(
        _router_body,
        grid=(ntb,),
        in_specs=[
            pl.BlockSpec((TB, D_MODEL), lambda t: (t, 0)),
            pl.BlockSpec((D_MODEL, NE), lambda t: (0, 0)),
        ],
        out_specs=[
            pl.BlockSpec((TB, 1), lambda t: (t, 0)),
            pl.BlockSpec((TB, 1), lambda t: (t, 0)),
            pl.BlockSpec((TB, L), lambda t: (t, 0)),
            pl.BlockSpec((TB, L), lambda t: (t, 0)),
            pl.BlockSpec((NE, SPB), lambda t: (0, 0)),
            pl.BlockSpec((NE, SPB), lambda t: (0, 0)),
        ],
        out_shape=[
            shp((T, 1), jnp.int32), shp((T, 1), jnp.int32),
            shp((T, L), jnp.float32), shp((T, L), jnp.float32),
            shp((NE, SPB), jnp.int32), shp((NE, SPB), jnp.int32),
        ],
        scratch_shapes=[pltpu.VMEM((1, NE), jnp.float32)],
    )(xf, gate_w)

    pos1 = p1o.reshape(T)
    pos2 = p2o.reshape(T)
    sorted_x = _make_dispatch()(xf, pos1, pos2)

    grid_spec = pltpu.PrefetchScalarGridSpec(
        num_scalar_prefetch=2,
        grid=(NB,),
        in_specs=[
            pl.BlockSpec((BLK, D_MODEL), lambda b, xm, val: (xm[b], 0)),
            pl.BlockSpec((1, D_MODEL, D_HID), lambda b, xm, val: (b // SPB, 0, 0)),
            pl.BlockSpec((1, 1, D_HID), lambda b, xm, val: (b // SPB, 0, 0)),
            pl.BlockSpec((1, D_HID, D_MODEL), lambda b, xm, val: (b // SPB, 0, 0)),
            pl.BlockSpec((1, 1, D_MODEL), lambda b, xm, val: (b // SPB, 0, 0)),
        ],
        out_specs=pl.BlockSpec((BLK, D_MODEL), lambda b, xm, val: (xm[b], 0)),
    )
    y_sorted = pl.pallas_call(
        _gemm_body,
        grid_spec=grid_spec,
        out_shape=shp((R, D_MODEL), jnp.float32),
    )(xmapo.reshape(NB), valo.reshape(NB), sorted_x, w1, b1, w2, b2)

    out = _make_combine()(y_sorted, pos1, pos2, wao, wbo)
    return out.reshape(B, T, D_MODEL)


# speedup vs baseline: 2.0892x; 1.0195x over previous
import functools

import jax
import jax.numpy as jnp
from jax import lax
from jax.experimental import pallas as pl
from jax.experimental.pallas import tpu as pltpu
from jax.experimental.pallas import tpu_sc as plsc

D_MODEL = 768
D_HID = 1536
NE = 8
TB = 512
BLK = 256
T = 2048
CAP = T
SPB = CAP // BLK
NB = NE * SPB
R = NE * CAP
NEG = -1e30

NW = 32
L = 16
TPW = T // NW
NCK = 4
CK = TPW // NCK


def _router_body(x_ref, gw_ref, p1_ref, p2_ref, wa_ref, wb_ref,
                 val_ref, xmap_ref, offs_ref):
    logits = jnp.dot(x_ref[...], gw_ref[...], preferred_element_type=jnp.float32)
    lane = lax.broadcasted_iota(jnp.int32, (TB, NE), 1)
    m1 = jnp.max(logits, axis=1, keepdims=True)
    i1 = jnp.min(jnp.where(logits == m1, lane, NE), axis=1, keepdims=True)
    l2 = jnp.where(lane == i1, NEG, logits)
    m2 = jnp.max(l2, axis=1, keepdims=True)
    i2 = jnp.min(jnp.where(l2 == m2, lane, NE), axis=1, keepdims=True)
    e2 = jnp.exp(m2 - m1)
    s = 1.0 + e2

    @pl.when(pl.program_id(0) == 0)
    def _():
        offs_ref[...] = jnp.zeros_like(offs_ref)

    pairmask = jnp.where((lane == i1) | (lane == i2), 1.0, 0.0)
    row = lax.broadcasted_iota(jnp.int32, (TB, TB), 0)
    col = lax.broadcasted_iota(jnp.int32, (TB, TB), 1)
    ltri = jnp.where(col < row, 1.0, 0.0)
    rank = jnp.dot(ltri, pairmask, preferred_element_type=jnp.float32)
    rank = rank + offs_ref[...]
    r1 = jnp.sum(jnp.where(lane == i1, rank, 0.0), axis=1,
                 keepdims=True).astype(jnp.int32)
    r2 = jnp.sum(jnp.where(lane == i2, rank, 0.0), axis=1,
                 keepdims=True).astype(jnp.int32)
    p1_ref[...] = i1 * CAP + r1
    p2_ref[...] = i2 * CAP + r2
    wa_ref[...] = jnp.broadcast_to(1.0 / s, (TB, L))
    wb_ref[...] = jnp.broadcast_to(e2 / s, (TB, L))
    offs = offs_ref[...] + jnp.sum(pairmask, axis=0, keepdims=True)
    offs_ref[...] = offs

    @pl.when(pl.program_id(0) == pl.num_programs(0) - 1)
    def _():
        nblk = jnp.floor((offs + (BLK - 1)) * (1.0 / BLK))
        siota = lax.broadcasted_iota(jnp.int32, (SPB, NE), 0).astype(jnp.float32)
        valid_c = jnp.where(siota * BLK < jnp.broadcast_to(offs, (SPB, NE)),
                            1.0, 0.0)
        smax = jnp.maximum(nblk - 1.0, 0.0)
        xmap_c = jnp.minimum(siota, jnp.broadcast_to(smax, (SPB, NE)))
        r8 = lax.broadcasted_iota(jnp.int32, (NE, NE), 0)
        c8 = lax.broadcasted_iota(jnp.int32, (NE, NE), 1)
        eye = jnp.where(r8 == c8, 1.0, 0.0)
        tdims = (((0,), (0,)), ((), ()))
        valid_r = lax.dot_general(valid_c, eye, tdims,
                                  preferred_element_type=jnp.float32)
        xmap_r = lax.dot_general(xmap_c, eye, tdims,
                                 preferred_element_type=jnp.float32)
        val_ref[...] = valid_r.astype(jnp.int32)
        xmap_ref[...] = (xmap_r + (r8 * SPB).astype(jnp.float32)).astype(jnp.int32)


def _make_dispatch():
    @functools.partial(
        pl.kernel,
        mesh=plsc.VectorSubcoreMesh(core_axis_name="c", subcore_axis_name="s"),
        out_type=jax.ShapeDtypeStruct((R, D_MODEL), jnp.float32),
        scratch_types=[
            pltpu.VMEM((TPW,), jnp.int32),
            pltpu.VMEM((TPW,), jnp.int32),
            pltpu.VMEM((TPW, D_MODEL), jnp.float32),
            pltpu.SemaphoreType.DMA,
            pltpu.SemaphoreType.DMA,
        ],
    )
    def dispatch(x_hbm, pos1_hbm, pos2_hbm, sx_hbm, idx1_v, idx2_v, rows_v,
                 sem1, sem2):
        wid = lax.axis_index("s") * 2 + lax.axis_index("c")
        base = wid * TPW
        pltpu.sync_copy(pos1_hbm.at[pl.ds(base, TPW)], idx1_v)
        pltpu.sync_copy(pos2_hbm.at[pl.ds(base, TPW)], idx2_v)
        pltpu.sync_copy(x_hbm.at[pl.ds(base, TPW)], rows_v)
        cp1 = pltpu.async_copy(rows_v, sx_hbm.at[idx1_v], sem1)
        cp2 = pltpu.async_copy(rows_v, sx_hbm.at[idx2_v], sem2)
        cp1.wait()
        cp2.wait()

    return dispatch


def _gemm_body(xmap_ref, val_ref, xs_ref, w1_ref, b1_ref, w2_ref, b2_ref, o_ref):
    b = pl.program_id(0)

    @pl.when(val_ref[b] != 0)
    def _():
        h = jnp.dot(xs_ref[...], w1_ref[0], preferred_element_type=jnp.float32)
        h = h + b1_ref[0]
        h = 0.5 * h * (1.0 + lax.erf(h * 0.7071067811865476))
        o_ref[...] = jnp.dot(h, w2_ref[0],
                             preferred_element_type=jnp.float32) + b2_ref[0]


def _make_combine():
    @functools.partial(
        pl.kernel,
        mesh=plsc.VectorSubcoreMesh(core_axis_name="c", subcore_axis_name="s"),
        out_type=jax.ShapeDtypeStruct((T, D_MODEL), jnp.float32),
        scratch_types=[
            pltpu.VMEM((TPW,), jnp.int32),
            pltpu.VMEM((TPW,), jnp.int32),
            pltpu.VMEM((TPW, L), jnp.float32),
            pltpu.VMEM((TPW, L), jnp.float32),
            pltpu.VMEM((TPW, D_MODEL), jnp.float32),
            pltpu.VMEM((TPW, D_MODEL), jnp.float32),
            pltpu.SemaphoreType.DMA,
            pltpu.SemaphoreType.DMA,
        ],
    )
    def combine(y_hbm, pos1_hbm, pos2_hbm, wa_hbm, wb_hbm, out_hbm,
                idx1_v, idx2_v, wa_v, wb_v, buf1, buf2, sem1, sem2):
        wid = lax.axis_index("s") * 2 + lax.axis_index("c")
        base = wid * TPW
        pltpu.sync_copy(pos1_hbm.at[pl.ds(base, TPW)], idx1_v)
        pltpu.sync_copy(pos2_hbm.at[pl.ds(base, TPW)], idx2_v)
        pltpu.sync_copy(wa_hbm.at[pl.ds(base, TPW)], wa_v)
        pltpu.sync_copy(wb_hbm.at[pl.ds(base, TPW)], wb_v)
        cp1 = pltpu.async_copy(y_hbm.at[idx1_v], buf1, sem1)
        cp2 = pltpu.async_copy(y_hbm.at[idx2_v], buf2, sem2)
        cp1.wait()
        cp2.wait()

        def body(t, carry):
            wa = wa_v[t, :]
            wb = wb_v[t, :]
            for j in range(D_MODEL // L):
                sl = pl.ds(j * L, L)
                buf1[t, sl] = wa * buf1[t, sl] + wb * buf2[t, sl]
            return carry

        lax.fori_loop(0, TPW, body, 0)
        pltpu.sync_copy(buf1, out_hbm.at[pl.ds(base, TPW)])

    return combine


def kernel(x, gate_w, w1, b1, w2, b2):
    B = x.shape[0]
    xf = x.reshape(T, D_MODEL)
    ntb = T // TB
    shp = jax.ShapeDtypeStruct

    p1o, p2o, wao, wbo, valo, xmapo = pl.pallas_call(
        _router_body,
        grid=(ntb,),
        in_specs=[
            pl.BlockSpec((TB, D_MODEL), lambda t: (t, 0)),
            pl.BlockSpec((D_MODEL, NE), lambda t: (0, 0)),
        ],
        out_specs=[
            pl.BlockSpec((TB, 1), lambda t: (t, 0)),
            pl.BlockSpec((TB, 1), lambda t: (t, 0)),
            pl.BlockSpec((TB, L), lambda t: (t, 0)),
            pl.BlockSpec((TB, L), lambda t: (t, 0)),
            pl.BlockSpec((NE, SPB), lambda t: (0, 0)),
            pl.BlockSpec((NE, SPB), lambda t: (0, 0)),
        ],
        out_shape=[
            shp((T, 1), jnp.int32), shp((T, 1), jnp.int32),
            shp((T, L), jnp.float32), shp((T, L), jnp.float32),
            shp((NE, SPB), jnp.int32), shp((NE, SPB), jnp.int32),
        ],
        scratch_shapes=[pltpu.VMEM((1, NE), jnp.float32)],
    )(xf, gate_w)

    pos1 = p1o.reshape(T)
    pos2 = p2o.reshape(T)
    sorted_x = _make_dispatch()(xf, pos1, pos2)

    grid_spec = pltpu.PrefetchScalarGridSpec(
        num_scalar_prefetch=2,
        grid=(NB,),
        in_specs=[
            pl.BlockSpec((BLK, D_MODEL), lambda b, xm, val: (xm[b], 0)),
            pl.BlockSpec((1, D_MODEL, D_HID), lambda b, xm, val: (b // SPB, 0, 0)),
            pl.BlockSpec((1, 1, D_HID), lambda b, xm, val: (b // SPB, 0, 0)),
            pl.BlockSpec((1, D_HID, D_MODEL), lambda b, xm, val: (b // SPB, 0, 0)),
            pl.BlockSpec((1, 1, D_MODEL), lambda b, xm, val: (b // SPB, 0, 0)),
        ],
        out_specs=pl.BlockSpec((BLK, D_MODEL), lambda b, xm, val: (xm[b], 0)),
    )
    y_sorted = pl.pallas_call(
        _gemm_body,
        grid_spec=grid_spec,
        out_shape=shp((R, D_MODEL), jnp.float32),
    )(xmapo.reshape(NB), valo.reshape(NB), sorted_x, w1, b1, w2, b2)

    out = _make_combine()(y_sorted, pos1, pos2, wao, wbo)
    return out.reshape(B, T, D_MODEL)
